# Initial kernel scaffold; baseline (speedup 1.0000x reference)
import functools

import jax
import jax.numpy as jnp
from jax import lax
from jax.experimental import pallas as pl
from jax.experimental.pallas import tpu as pltpu
from jax.experimental.pallas import tpu_sc as plsc

N, E, D = 10000, 320000, 128
NC, NS = 2, 16
NW = NC * NS
EPT = E // NW
CHUNK = 80
NCHUNK = EPT // CHUNK
ROWS_PER_SUB = N // NS
ZROWS = 125
DEGW = 16


def _sc_aggregate(x, edge_index):
    mesh = plsc.VectorSubcoreMesh(
        core_axis_name="c", subcore_axis_name="s", num_cores=NC, num_subcores=NS
    )

    @functools.partial(
        pl.kernel,
        out_type=[
            jax.ShapeDtypeStruct((NC, N, D), jnp.float32),
            jax.ShapeDtypeStruct((NC, N, DEGW), jnp.float32),
        ],
        mesh=mesh,
        scratch_types=[
            pltpu.VMEM((CHUNK,), jnp.int32),
            pltpu.VMEM((CHUNK,), jnp.int32),
            pltpu.VMEM((CHUNK, D), jnp.float32),
            pltpu.VMEM((CHUNK, DEGW), jnp.float32),
            pltpu.VMEM((ZROWS, D), jnp.float32),
            pltpu.VMEM((ZROWS, DEGW), jnp.float32),
            pltpu.VMEM_SHARED((N, D), jnp.float32),
            pltpu.VMEM_SHARED((N, DEGW), jnp.float32),
            pltpu.SemaphoreType.DMA,
        ],
    )
    def k(x_hbm, e_hbm, out_hbm, deg_hbm, src_v, dst_v, rows_v, ones_v,
          z_v, zd_v, acc_sh, deg_sh, sem):
        cid = lax.axis_index("c")
        sid = lax.axis_index("s")
        wid = cid * NS + sid

        zero16 = jnp.zeros((16,), jnp.float32)
        one16 = jnp.ones((16,), jnp.float32)

        @pl.loop(0, ZROWS)
        def _(r):
            zd_v[r, :] = zero16

            @pl.loop(0, D, step=16)
            def _(cc):
                z_v[r, pl.ds(cc, 16)] = zero16

        @pl.loop(0, CHUNK)
        def _(r):
            ones_v[r, :] = one16

        @pl.loop(0, ROWS_PER_SUB // ZROWS)
        def _(kk):
            base = sid * ROWS_PER_SUB + kk * ZROWS
            pltpu.sync_copy(z_v, acc_sh.at[pl.ds(base, ZROWS)])
            pltpu.sync_copy(zd_v, deg_sh.at[pl.ds(base, ZROWS)])

        plsc.subcore_barrier()

        ebase = wid * EPT

        @pl.loop(0, NCHUNK)
        def _(j):
            b = ebase + j * CHUNK
            pltpu.sync_copy(e_hbm.at[0, pl.ds(b, CHUNK)], src_v)
            pltpu.sync_copy(e_hbm.at[1, pl.ds(b, CHUNK)], dst_v)
            pltpu.async_copy(x_hbm.at[src_v], rows_v, sem).wait()
            pltpu.sync_copy(rows_v, acc_sh.at[dst_v], add=True)
            pltpu.sync_copy(ones_v, deg_sh.at[dst_v], add=True)

        plsc.subcore_barrier()

        @pl.loop(0, ROWS_PER_SUB // ZROWS)
        def _(kk):
            base = sid * ROWS_PER_SUB + kk * ZROWS
            pltpu.sync_copy(acc_sh.at[pl.ds(base, ZROWS)],
                            out_hbm.at[cid, pl.ds(base, ZROWS)])
            pltpu.sync_copy(deg_sh.at[pl.ds(base, ZROWS)],
                            deg_hbm.at[cid, pl.ds(base, ZROWS)])

    return k(x, edge_index)


BLK = 2000


def _tc_combine(x, part, degp, W_self, W_neigh, b, gamma, beta):
    def body(x_ref, p_ref, d_ref, ws_ref, wn_ref, b_ref, g_ref, be_ref, o_ref):
        xb = x_ref[...]
        psum = p_ref[0] + p_ref[1]
        deg = d_ref[0, :, 0:1] + d_ref[1, :, 0:1]
        agg = psum / jnp.maximum(deg, 1.0)
        h = jnp.dot(xb, ws_ref[...], preferred_element_type=jnp.float32)
        h = h + jnp.dot(agg, wn_ref[...], preferred_element_type=jnp.float32)
        h = h + b_ref[...]
        mu = jnp.mean(h, axis=1, keepdims=True)
        var = jnp.mean((h - mu) * (h - mu), axis=1, keepdims=True)
        h = (h - mu) * lax.rsqrt(var + 1e-5) * g_ref[...] + be_ref[...]
        o_ref[...] = jnp.maximum(h, 0.0) + xb

    return pl.pallas_call(
        body,
        grid=(N // BLK,),
        in_specs=[
            pl.BlockSpec((BLK, D), lambda i: (i, 0)),
            pl.BlockSpec((NC, BLK, D), lambda i: (0, i, 0)),
            pl.BlockSpec((NC, BLK, DEGW), lambda i: (0, i, 0)),
            pl.BlockSpec((D, D), lambda i: (0, 0)),
            pl.BlockSpec((D, D), lambda i: (0, 0)),
            pl.BlockSpec((1, D), lambda i: (0, 0)),
            pl.BlockSpec((1, D), lambda i: (0, 0)),
            pl.BlockSpec((1, D), lambda i: (0, 0)),
        ],
        out_specs=pl.BlockSpec((BLK, D), lambda i: (i, 0)),
        out_shape=jax.ShapeDtypeStruct((N, D), jnp.float32),
    )(x, part, degp, W_self, W_neigh, b, gamma, beta)


@jax.jit
def kernel(x, edge_index, W_self, W_neigh, b, gamma, beta):
    part, degp = _sc_aggregate(x, edge_index)
    return _tc_combine(
        x, part, degp, W_self, W_neigh,
        b.reshape(1, D), gamma.reshape(1, D), beta.reshape(1, D),
    )


# speedup vs baseline: 3.9899x; 3.9899x over previous
import functools

import jax
import jax.numpy as jnp
from jax import lax
from jax.experimental import pallas as pl
from jax.experimental.pallas import tpu as pltpu
from jax.experimental.pallas import tpu_sc as plsc

N, E, D = 10000, 320000, 128
HALF = D // 2
NC, NS = 2, 16
EPT = E // NS
CHUNK = 80
NCHUNK = EPT // CHUNK
NPAD = 10240
ROWS_PER_SUB = NPAD // NS
DEGW = 16


def _sc_aggregate(xs, edge_flat):
    mesh = plsc.VectorSubcoreMesh(
        core_axis_name="c", subcore_axis_name="s", num_cores=NC, num_subcores=NS
    )

    @functools.partial(
        pl.kernel,
        out_type=[
            jax.ShapeDtypeStruct((NC, NPAD, HALF), jnp.float32),
            jax.ShapeDtypeStruct((NPAD, DEGW), jnp.float32),
        ],
        mesh=mesh,
        scratch_types=[
            pltpu.VMEM((CHUNK,), jnp.int32),
            pltpu.VMEM((CHUNK,), jnp.int32),
            pltpu.VMEM((CHUNK, HALF), jnp.float32),
            pltpu.VMEM((CHUNK, DEGW), jnp.float32),
            pltpu.VMEM((CHUNK, DEGW), jnp.float32),
            pltpu.VMEM_SHARED((NPAD, HALF), jnp.float32),
            pltpu.VMEM_SHARED((NPAD, DEGW), jnp.float32),
            pltpu.SemaphoreType.DMA,
        ],
        compiler_params=pltpu.CompilerParams(use_tc_tiling_on_sc=False),
    )
    def k(xs_hbm, e_hbm, out_hbm, deg_hbm, src_v, dst_v, rows_v, ones_v,
          zd_v, acc_sh, deg_sh, sem):
        cid = lax.axis_index("c")
        sid = lax.axis_index("s")

        zero16 = jnp.zeros((16,), jnp.float32)
        one16 = jnp.ones((16,), jnp.float32)

        @pl.loop(0, CHUNK)
        def _(r):
            ones_v[r, :] = one16
            zd_v[r, :] = zero16

            @pl.loop(0, HALF, step=16)
            def _(cc):
                rows_v[r, pl.ds(cc, 16)] = zero16

        @pl.loop(0, ROWS_PER_SUB // CHUNK)
        def _(kk):
            base = sid * ROWS_PER_SUB + kk * CHUNK
            pltpu.sync_copy(rows_v, acc_sh.at[pl.ds(base, CHUNK)])

            @pl.when(cid == 0)
            def _():
                pltpu.sync_copy(zd_v, deg_sh.at[pl.ds(base, CHUNK)])

        plsc.subcore_barrier()

        ebase = sid * EPT
        xh = xs_hbm.at[cid]

        @pl.loop(0, NCHUNK)
        def _(j):
            b = ebase + j * CHUNK
            pltpu.sync_copy(e_hbm.at[pl.ds(b, CHUNK)], src_v)
            pltpu.sync_copy(e_hbm.at[pl.ds(E + b, CHUNK)], dst_v)
            pltpu.async_copy(xh.at[src_v], rows_v, sem).wait()
            pltpu.sync_copy(rows_v, acc_sh.at[dst_v], add=True)

            @pl.when(cid == 0)
            def _():
                pltpu.sync_copy(ones_v, deg_sh.at[dst_v], add=True)

        plsc.subcore_barrier()

        base = sid * ROWS_PER_SUB
        pltpu.sync_copy(acc_sh.at[pl.ds(base, ROWS_PER_SUB)],
                        out_hbm.at[cid, pl.ds(base, ROWS_PER_SUB)])

        @pl.when(cid == 0)
        def _():
            pltpu.sync_copy(deg_sh.at[pl.ds(base, ROWS_PER_SUB)],
                            deg_hbm.at[pl.ds(base, ROWS_PER_SUB)])

    return k(xs, edge_flat)


BLK = 2000


def _tc_combine(x, part, degp, W_self, W_neigh, b, gamma, beta):
    def body(x_ref, p_ref, d_ref, ws_ref, wn_ref, b_ref, g_ref, be_ref, o_ref):
        xb = x_ref[...]
        psum = jnp.concatenate([p_ref[0], p_ref[1]], axis=1)
        deg = d_ref[:, 0:1]
        agg = psum / jnp.maximum(deg, 1.0)
        h = jnp.dot(xb, ws_ref[...], preferred_element_type=jnp.float32)
        h = h + jnp.dot(agg, wn_ref[...], preferred_element_type=jnp.float32)
        h = h + b_ref[...]
        mu = jnp.mean(h, axis=1, keepdims=True)
        var = jnp.mean((h - mu) * (h - mu), axis=1, keepdims=True)
        h = (h - mu) * lax.rsqrt(var + 1e-5) * g_ref[...] + be_ref[...]
        o_ref[...] = jnp.maximum(h, 0.0) + xb

    return pl.pallas_call(
        body,
        grid=(N // BLK,),
        in_specs=[
            pl.BlockSpec((BLK, D), lambda i: (i, 0)),
            pl.BlockSpec((NC, BLK, HALF), lambda i: (0, i, 0)),
            pl.BlockSpec((BLK, DEGW), lambda i: (i, 0)),
            pl.BlockSpec((D, D), lambda i: (0, 0)),
            pl.BlockSpec((D, D), lambda i: (0, 0)),
            pl.BlockSpec((1, D), lambda i: (0, 0)),
            pl.BlockSpec((1, D), lambda i: (0, 0)),
            pl.BlockSpec((1, D), lambda i: (0, 0)),
        ],
        out_specs=pl.BlockSpec((BLK, D), lambda i: (i, 0)),
        out_shape=jax.ShapeDtypeStruct((N, D), jnp.float32),
    )(x, part, degp, W_self, W_neigh, b, gamma, beta)


@jax.jit
def kernel(x, edge_index, W_self, W_neigh, b, gamma, beta):
    xs = jnp.stack([x[:, :HALF], x[:, HALF:]])
    part, degp = _sc_aggregate(xs, edge_index.reshape(2 * E))
    return _tc_combine(
        x, part, degp, W_self, W_neigh,
        b.reshape(1, D), gamma.reshape(1, D), beta.reshape(1, D),
    )


# speedup vs baseline: 7.3331x; 1.8379x over previous
import functools

import jax
import jax.numpy as jnp
from jax import lax
from jax.experimental import pallas as pl
from jax.experimental.pallas import tpu as pltpu
from jax.experimental.pallas import tpu_sc as plsc

N, E, D = 10000, 320000, 128
HALF = D // 2
NC, NS = 2, 16
EPT = E // NS
CHUNK = 80
IB = 10
NBATCH = EPT // (CHUNK * IB)
NCHTOT = E // CHUNK
NPAD = 10240
ROWS_PER_SUB = NPAD // NS
DEGW = 16


def _sc_aggregate(xs, edges):
    mesh = plsc.VectorSubcoreMesh(
        core_axis_name="c", subcore_axis_name="s", num_cores=NC, num_subcores=NS
    )

    @functools.partial(
        pl.kernel,
        out_type=[
            jax.ShapeDtypeStruct((NC, NPAD, HALF), jnp.float32),
            jax.ShapeDtypeStruct((NC, NPAD, DEGW), jnp.float32),
        ],
        mesh=mesh,
        scratch_types=[
            pltpu.VMEM((IB, CHUNK), jnp.int32),
            pltpu.VMEM((IB, CHUNK), jnp.int32),
            pltpu.VMEM((CHUNK, HALF), jnp.float32),
            pltpu.VMEM((CHUNK, HALF), jnp.float32),
            pltpu.VMEM((CHUNK, DEGW), jnp.float32),
            pltpu.VMEM((CHUNK, DEGW), jnp.float32),
            pltpu.VMEM_SHARED((NPAD, HALF), jnp.float32),
            pltpu.VMEM_SHARED((NPAD, DEGW), jnp.float32),
            pltpu.SemaphoreType.DMA,
            pltpu.SemaphoreType.DMA,
        ],
        compiler_params=pltpu.CompilerParams(use_tc_tiling_on_sc=False),
    )
    def k(xs_hbm, e_hbm, out_hbm, deg_hbm, src_v, dst_v, rows0, rows1,
          ones_v, zd_v, acc_sh, deg_sh, sem0, sem1):
        cid = lax.axis_index("c")
        sid = lax.axis_index("s")

        zero16 = jnp.zeros((16,), jnp.float32)
        one16 = jnp.ones((16,), jnp.float32)

        @pl.loop(0, CHUNK)
        def _(r):
            ones_v[r, :] = one16
            zd_v[r, :] = zero16

            @pl.loop(0, HALF, step=16)
            def _(cc):
                rows0[r, pl.ds(cc, 16)] = zero16

        @pl.loop(0, ROWS_PER_SUB // CHUNK)
        def _(kk):
            base = sid * ROWS_PER_SUB + kk * CHUNK
            pltpu.sync_copy(rows0, acc_sh.at[pl.ds(base, CHUNK)])
            pltpu.sync_copy(zd_v, deg_sh.at[pl.ds(base, CHUNK)])

        plsc.subcore_barrier()

        xh = xs_hbm.at[cid]
        rows = (rows0, rows1)
        sems = (sem0, sem1)
        cbase = sid * (NBATCH * IB)

        @pl.loop(0, NBATCH)
        def _(g):
            cb = cbase + g * IB
            pltpu.sync_copy(e_hbm.at[0, pl.ds(cb, IB)], src_v)
            pltpu.sync_copy(e_hbm.at[1, pl.ds(cb, IB)], dst_v)
            copies = [None, None]
            copies[0] = pltpu.async_copy(xh.at[src_v.at[0]], rows0, sem0)
            for c in range(IB):
                copies[c % 2].wait()
                if c + 1 < IB:
                    copies[(c + 1) % 2] = pltpu.async_copy(
                        xh.at[src_v.at[c + 1]], rows[(c + 1) % 2],
                        sems[(c + 1) % 2])
                pltpu.sync_copy(rows[c % 2], acc_sh.at[dst_v.at[c]], add=True)
                deg_core = c % 2

                @pl.when(cid == deg_core)
                def _():
                    pltpu.sync_copy(ones_v, deg_sh.at[dst_v.at[c]], add=True)

        plsc.subcore_barrier()

        base = sid * ROWS_PER_SUB
        pltpu.sync_copy(acc_sh.at[pl.ds(base, ROWS_PER_SUB)],
                        out_hbm.at[cid, pl.ds(base, ROWS_PER_SUB)])
        pltpu.sync_copy(deg_sh.at[pl.ds(base, ROWS_PER_SUB)],
                        deg_hbm.at[cid, pl.ds(base, ROWS_PER_SUB)])

    return k(xs, edges)


BLK = 2000


def _tc_combine(x, part, degp, W_self, W_neigh, b, gamma, beta):
    def body(x_ref, p_ref, d_ref, ws_ref, wn_ref, b_ref, g_ref, be_ref, o_ref):
        xb = x_ref[...]
        psum = jnp.concatenate([p_ref[0], p_ref[1]], axis=1)
        deg = d_ref[0, :, 0:1] + d_ref[1, :, 0:1]
        agg = psum / jnp.maximum(deg, 1.0)
        h = jnp.dot(xb, ws_ref[...], preferred_element_type=jnp.float32)
        h = h + jnp.dot(agg, wn_ref[...], preferred_element_type=jnp.float32)
        h = h + b_ref[...]
        mu = jnp.mean(h, axis=1, keepdims=True)
        var = jnp.mean((h - mu) * (h - mu), axis=1, keepdims=True)
        h = (h - mu) * lax.rsqrt(var + 1e-5) * g_ref[...] + be_ref[...]
        o_ref[...] = jnp.maximum(h, 0.0) + xb

    return pl.pallas_call(
        body,
        grid=(N // BLK,),
        in_specs=[
            pl.BlockSpec((BLK, D), lambda i: (i, 0)),
            pl.BlockSpec((NC, BLK, HALF), lambda i: (0, i, 0)),
            pl.BlockSpec((NC, BLK, DEGW), lambda i: (0, i, 0)),
            pl.BlockSpec((D, D), lambda i: (0, 0)),
            pl.BlockSpec((D, D), lambda i: (0, 0)),
            pl.BlockSpec((1, D), lambda i: (0, 0)),
            pl.BlockSpec((1, D), lambda i: (0, 0)),
            pl.BlockSpec((1, D), lambda i: (0, 0)),
        ],
        out_specs=pl.BlockSpec((BLK, D), lambda i: (i, 0)),
        out_shape=jax.ShapeDtypeStruct((N, D), jnp.float32),
    )(x, part, degp, W_self, W_neigh, b, gamma, beta)


@jax.jit
def kernel(x, edge_index, W_self, W_neigh, b, gamma, beta):
    xs = jnp.stack([x[:, :HALF], x[:, HALF:]])
    edges = edge_index.reshape(2, NCHTOT, CHUNK)
    part, degp = _sc_aggregate(xs, edges)
    return _tc_combine(
        x, part, degp, W_self, W_neigh,
        b.reshape(1, D), gamma.reshape(1, D), beta.reshape(1, D),
    )


# speedup vs baseline: 8.8241x; 1.2033x over previous
import functools

import jax
import jax.numpy as jnp
from jax import lax
from jax.experimental import pallas as pl
from jax.experimental.pallas import tpu as pltpu
from jax.experimental.pallas import tpu_sc as plsc

N, E, D = 10000, 320000, 128
HALF = D // 2
NC, NS = 2, 16
EPT = E // NS
CHUNK = 125
IB = 10
NBATCH = EPT // (CHUNK * IB)
NCHTOT = E // CHUNK
ZR = 80
NPAD = 10240
ROWS_PER_SUB = NPAD // NS
DEGW = 16


def _sc_aggregate(xs, edges):
    mesh = plsc.VectorSubcoreMesh(
        core_axis_name="c", subcore_axis_name="s", num_cores=NC, num_subcores=NS
    )

    @functools.partial(
        pl.kernel,
        out_type=[
            jax.ShapeDtypeStruct((NC, NPAD, HALF), jnp.float32),
            jax.ShapeDtypeStruct((NC, NPAD, DEGW), jnp.float32),
        ],
        mesh=mesh,
        scratch_types=[
            pltpu.VMEM((IB, CHUNK), jnp.int32),
            pltpu.VMEM((IB, CHUNK), jnp.int32),
            pltpu.VMEM((CHUNK, HALF), jnp.float32),
            pltpu.VMEM((CHUNK, HALF), jnp.float32),
            pltpu.VMEM((CHUNK, DEGW), jnp.float32),
            pltpu.VMEM((CHUNK, DEGW), jnp.float32),
            pltpu.VMEM_SHARED((NPAD, HALF), jnp.float32),
            pltpu.VMEM_SHARED((NPAD, DEGW), jnp.float32),
            pltpu.SemaphoreType.DMA,
            pltpu.SemaphoreType.DMA,
            pltpu.SemaphoreType.DMA,
            pltpu.SemaphoreType.DMA,
            pltpu.SemaphoreType.DMA,
        ],
        compiler_params=pltpu.CompilerParams(use_tc_tiling_on_sc=False),
    )
    def k(xs_hbm, e_hbm, out_hbm, deg_hbm, src_v, dst_v, rows0, rows1,
          ones_v, zd_v, acc_sh, deg_sh, gsem0, gsem1, ssem0, ssem1, dsem):
        cid = lax.axis_index("c")
        sid = lax.axis_index("s")

        zero16 = jnp.zeros((16,), jnp.float32)
        one16 = jnp.ones((16,), jnp.float32)

        @pl.loop(0, CHUNK)
        def _(r):
            ones_v[r, :] = one16
            zd_v[r, :] = zero16

            @pl.loop(0, HALF, step=16)
            def _(cc):
                rows0[r, pl.ds(cc, 16)] = zero16

        @pl.loop(0, ROWS_PER_SUB // ZR)
        def _(kk):
            base = sid * ROWS_PER_SUB + kk * ZR
            pltpu.sync_copy(rows0.at[pl.ds(0, ZR)], acc_sh.at[pl.ds(base, ZR)])
            pltpu.sync_copy(zd_v.at[pl.ds(0, ZR)], deg_sh.at[pl.ds(base, ZR)])

        plsc.subcore_barrier()

        xh = xs_hbm.at[cid]
        rows = (rows0, rows1)
        gsems = (gsem0, gsem1)
        ssems = (ssem0, ssem1)
        cbase = sid * (NBATCH * IB)

        @pl.loop(0, NBATCH)
        def _(g):
            cb = cbase + g * IB
            pltpu.sync_copy(e_hbm.at[0, pl.ds(cb, IB)], src_v)
            pltpu.sync_copy(e_hbm.at[1, pl.ds(cb, IB)], dst_v)
            gat = [None, None]
            scats = [None, None]
            deg_descs = []
            gat[0] = pltpu.async_copy(xh.at[src_v.at[0]], rows0, gsem0)
            for c in range(IB):
                b = c % 2
                gat[b].wait()
                if c + 1 < IB:
                    nb = (c + 1) % 2
                    if scats[nb] is not None:
                        scats[nb].wait()
                    gat[nb] = pltpu.async_copy(
                        xh.at[src_v.at[c + 1]], rows[nb], gsems[nb])
                scats[b] = pltpu.async_copy(
                    rows[b], acc_sh.at[dst_v.at[c]], ssems[b], add=True)
                deg_descs.append(pltpu.async_copy(
                    ones_v, deg_sh.at[dst_v.at[c]], dsem, add=True))
            for sc in scats:
                if sc is not None:
                    sc.wait()
            for dd in deg_descs:
                dd.wait()

        plsc.subcore_barrier()

        base = sid * ROWS_PER_SUB
        pltpu.sync_copy(acc_sh.at[pl.ds(base, ROWS_PER_SUB)],
                        out_hbm.at[cid, pl.ds(base, ROWS_PER_SUB)])
        pltpu.sync_copy(deg_sh.at[pl.ds(base, ROWS_PER_SUB)],
                        deg_hbm.at[cid, pl.ds(base, ROWS_PER_SUB)])

    return k(xs, edges)


BLK = 2000


def _tc_combine(x, part, degp, W_self, W_neigh, b, gamma, beta):
    def body(x_ref, p_ref, d_ref, ws_ref, wn_ref, b_ref, g_ref, be_ref, o_ref):
        xb = x_ref[...]
        psum = jnp.concatenate([p_ref[0], p_ref[1]], axis=1)
        deg = (d_ref[0, :, 0:1] + d_ref[1, :, 0:1]) * 0.5
        agg = psum / jnp.maximum(deg, 1.0)
        h = jnp.dot(xb, ws_ref[...], preferred_element_type=jnp.float32)
        h = h + jnp.dot(agg, wn_ref[...], preferred_element_type=jnp.float32)
        h = h + b_ref[...]
        mu = jnp.mean(h, axis=1, keepdims=True)
        var = jnp.mean((h - mu) * (h - mu), axis=1, keepdims=True)
        h = (h - mu) * lax.rsqrt(var + 1e-5) * g_ref[...] + be_ref[...]
        o_ref[...] = jnp.maximum(h, 0.0) + xb

    return pl.pallas_call(
        body,
        grid=(N // BLK,),
        in_specs=[
            pl.BlockSpec((BLK, D), lambda i: (i, 0)),
            pl.BlockSpec((NC, BLK, HALF), lambda i: (0, i, 0)),
            pl.BlockSpec((NC, BLK, DEGW), lambda i: (0, i, 0)),
            pl.BlockSpec((D, D), lambda i: (0, 0)),
            pl.BlockSpec((D, D), lambda i: (0, 0)),
            pl.BlockSpec((1, D), lambda i: (0, 0)),
            pl.BlockSpec((1, D), lambda i: (0, 0)),
            pl.BlockSpec((1, D), lambda i: (0, 0)),
        ],
        out_specs=pl.BlockSpec((BLK, D), lambda i: (i, 0)),
        out_shape=jax.ShapeDtypeStruct((N, D), jnp.float32),
    )(x, part, degp, W_self, W_neigh, b, gamma, beta)


@jax.jit
def kernel(x, edge_index, W_self, W_neigh, b, gamma, beta):
    xs = jnp.stack([x[:, :HALF], x[:, HALF:]])
    edges = edge_index.reshape(2, NCHTOT, CHUNK)
    part, degp = _sc_aggregate(xs, edges)
    return _tc_combine(
        x, part, degp, W_self, W_neigh,
        b.reshape(1, D), gamma.reshape(1, D), beta.reshape(1, D),
    )


# speedup vs baseline: 11.2802x; 1.2783x over previous
import functools

import jax
import jax.numpy as jnp
from jax import lax
from jax.experimental import pallas as pl
from jax.experimental.pallas import tpu as pltpu
from jax.experimental.pallas import tpu_sc as plsc

N, E, D = 10000, 320000, 128
HALF = D // 2
NC, NS = 2, 16
EPT = E // NS
CHUNK = 125
IB = 10
NBATCH = EPT // (CHUNK * IB)
NCHTOT = E // CHUNK
ZR = 80
NPAD = 10240
ROWS_PER_SUB = NPAD // NS
DEGW = 16


def _sc_aggregate(xs, edges):
    mesh = plsc.VectorSubcoreMesh(
        core_axis_name="c", subcore_axis_name="s", num_cores=NC, num_subcores=NS
    )

    @functools.partial(
        pl.kernel,
        out_type=[
            jax.ShapeDtypeStruct((NC, NPAD, HALF), jnp.float32),
            jax.ShapeDtypeStruct((NC, NPAD, DEGW), jnp.float32),
        ],
        mesh=mesh,
        scratch_types=[
            pltpu.VMEM((IB, CHUNK), jnp.int32),
            pltpu.VMEM((IB, CHUNK), jnp.int32),
            pltpu.VMEM((CHUNK, HALF), jnp.float32),
            pltpu.VMEM((CHUNK, HALF), jnp.float32),
            pltpu.VMEM((CHUNK, HALF), jnp.float32),
            pltpu.VMEM((CHUNK, DEGW), jnp.float32),
            pltpu.VMEM((CHUNK, DEGW), jnp.float32),
            pltpu.VMEM_SHARED((NPAD, HALF), jnp.float32),
            pltpu.VMEM_SHARED((NPAD, DEGW), jnp.float32),
            pltpu.SemaphoreType.DMA,
            pltpu.SemaphoreType.DMA,
            pltpu.SemaphoreType.DMA,
            pltpu.SemaphoreType.DMA,
            pltpu.SemaphoreType.DMA,
            pltpu.SemaphoreType.DMA,
            pltpu.SemaphoreType.DMA,
        ],
        compiler_params=pltpu.CompilerParams(use_tc_tiling_on_sc=False),
    )
    def k(xs_hbm, e_hbm, out_hbm, deg_hbm, src_v, dst_v, rows0, rows1, rows2,
          ones_v, zd_v, acc_sh, deg_sh, gsem0, gsem1, gsem2,
          ssem0, ssem1, ssem2, dsem):
        cid = lax.axis_index("c")
        sid = lax.axis_index("s")

        zero16 = jnp.zeros((16,), jnp.float32)
        one16 = jnp.ones((16,), jnp.float32)

        @pl.loop(0, CHUNK)
        def _(r):
            ones_v[r, :] = one16
            zd_v[r, :] = zero16

            @pl.loop(0, HALF, step=16)
            def _(cc):
                rows0[r, pl.ds(cc, 16)] = zero16

        @pl.loop(0, ROWS_PER_SUB // ZR)
        def _(kk):
            base = sid * ROWS_PER_SUB + kk * ZR
            pltpu.sync_copy(rows0.at[pl.ds(0, ZR)], acc_sh.at[pl.ds(base, ZR)])
            pltpu.sync_copy(zd_v.at[pl.ds(0, ZR)], deg_sh.at[pl.ds(base, ZR)])

        plsc.subcore_barrier()

        xh = xs_hbm.at[cid]
        rows = (rows0, rows1, rows2)
        gsems = (gsem0, gsem1, gsem2)
        ssems = (ssem0, ssem1, ssem2)
        cbase = sid * (NBATCH * IB)

        @pl.loop(0, NBATCH)
        def _(g):
            cb = cbase + g * IB
            pltpu.sync_copy(e_hbm.at[0, pl.ds(cb, IB)], src_v)
            pltpu.sync_copy(e_hbm.at[1, pl.ds(cb, IB)], dst_v)
            gat = [None, None, None]
            scats = [None, None, None]
            deg_descs = []
            gat[0] = pltpu.async_copy(xh.at[src_v.at[0]], rows0, gsem0)
            gat[1] = pltpu.async_copy(xh.at[src_v.at[1]], rows1, gsem1)
            for c in range(IB):
                b = c % 3
                gat[b].wait()
                if c + 2 < IB:
                    nb = (c + 2) % 3
                    if scats[nb] is not None:
                        scats[nb].wait()
                    gat[nb] = pltpu.async_copy(
                        xh.at[src_v.at[c + 2]], rows[nb], gsems[nb])
                scats[b] = pltpu.async_copy(
                    rows[b], acc_sh.at[dst_v.at[c]], ssems[b], add=True)
                deg_descs.append(pltpu.async_copy(
                    ones_v, deg_sh.at[dst_v.at[c]], dsem, add=True))
            for sc in scats:
                if sc is not None:
                    sc.wait()
            for dd in deg_descs:
                dd.wait()

        plsc.subcore_barrier()

        base = sid * ROWS_PER_SUB
        pltpu.sync_copy(acc_sh.at[pl.ds(base, ROWS_PER_SUB)],
                        out_hbm.at[cid, pl.ds(base, ROWS_PER_SUB)])
        pltpu.sync_copy(deg_sh.at[pl.ds(base, ROWS_PER_SUB)],
                        deg_hbm.at[cid, pl.ds(base, ROWS_PER_SUB)])

    return k(xs, edges)


BLK = 2000


def _tc_combine(x, part, degp, W_self, W_neigh, b, gamma, beta):
    def body(x_ref, p_ref, d_ref, ws_ref, wn_ref, b_ref, g_ref, be_ref, o_ref):
        xb = x_ref[...]
        psum = jnp.concatenate([p_ref[0], p_ref[1]], axis=1)
        deg = (d_ref[0, :, 0:1] + d_ref[1, :, 0:1]) * 0.5
        agg = psum / jnp.maximum(deg, 1.0)
        h = jnp.dot(xb, ws_ref[...], preferred_element_type=jnp.float32)
        h = h + jnp.dot(agg, wn_ref[...], preferred_element_type=jnp.float32)
        h = h + b_ref[...]
        mu = jnp.mean(h, axis=1, keepdims=True)
        var = jnp.mean((h - mu) * (h - mu), axis=1, keepdims=True)
        h = (h - mu) * lax.rsqrt(var + 1e-5) * g_ref[...] + be_ref[...]
        o_ref[...] = jnp.maximum(h, 0.0) + xb

    return pl.pallas_call(
        body,
        grid=(N // BLK,),
        in_specs=[
            pl.BlockSpec((BLK, D), lambda i: (i, 0)),
            pl.BlockSpec((NC, BLK, HALF), lambda i: (0, i, 0)),
            pl.BlockSpec((NC, BLK, DEGW), lambda i: (0, i, 0)),
            pl.BlockSpec((D, D), lambda i: (0, 0)),
            pl.BlockSpec((D, D), lambda i: (0, 0)),
            pl.BlockSpec((1, D), lambda i: (0, 0)),
            pl.BlockSpec((1, D), lambda i: (0, 0)),
            pl.BlockSpec((1, D), lambda i: (0, 0)),
        ],
        out_specs=pl.BlockSpec((BLK, D), lambda i: (i, 0)),
        out_shape=jax.ShapeDtypeStruct((N, D), jnp.float32),
    )(x, part, degp, W_self, W_neigh, b, gamma, beta)


@jax.jit
def kernel(x, edge_index, W_self, W_neigh, b, gamma, beta):
    xs = jnp.stack([x[:, :HALF], x[:, HALF:]])
    edges = edge_index.reshape(2, NCHTOT, CHUNK)
    part, degp = _sc_aggregate(xs, edges)
    return _tc_combine(
        x, part, degp, W_self, W_neigh,
        b.reshape(1, D), gamma.reshape(1, D), beta.reshape(1, D),
    )


# speedup vs baseline: 12.3503x; 1.0949x over previous
import functools

import jax
import jax.numpy as jnp
from jax import lax
from jax.experimental import pallas as pl
from jax.experimental.pallas import tpu as pltpu
from jax.experimental.pallas import tpu_sc as plsc

N, E, D = 10000, 320000, 128
HALF = D // 2
NC, NS = 2, 16
EPT = E // NS
CHUNK = 125
IB = 10
NBATCH = EPT // (CHUNK * IB)
NCHTOT = E // CHUNK
ZR = 80
NPAD = 10240
ROWS_PER_SUB = NPAD // NS
DEGW = 16


def _sc_aggregate(xs, edges):
    mesh = plsc.VectorSubcoreMesh(
        core_axis_name="c", subcore_axis_name="s", num_cores=NC, num_subcores=NS
    )

    @functools.partial(
        pl.kernel,
        out_type=[
            jax.ShapeDtypeStruct((NC, NPAD, HALF), jnp.float32),
            jax.ShapeDtypeStruct((NC, NPAD, DEGW), jnp.float32),
        ],
        mesh=mesh,
        scratch_types=[
            pltpu.VMEM((IB, CHUNK), jnp.int32),
            pltpu.VMEM((IB, CHUNK), jnp.int32),
            pltpu.VMEM((IB, CHUNK), jnp.int32),
            pltpu.VMEM((IB, CHUNK), jnp.int32),
            pltpu.VMEM((CHUNK, HALF), jnp.float32),
            pltpu.VMEM((CHUNK, HALF), jnp.float32),
            pltpu.VMEM((CHUNK, HALF), jnp.float32),
            pltpu.VMEM((CHUNK, DEGW), jnp.float32),
            pltpu.VMEM((CHUNK, DEGW), jnp.float32),
            pltpu.VMEM_SHARED((NPAD, HALF), jnp.float32),
            pltpu.VMEM_SHARED((NPAD, DEGW), jnp.float32),
            pltpu.SemaphoreType.DMA,
            pltpu.SemaphoreType.DMA,
            pltpu.SemaphoreType.DMA,
            pltpu.SemaphoreType.DMA,
            pltpu.SemaphoreType.DMA,
            pltpu.SemaphoreType.DMA,
            pltpu.SemaphoreType.DMA,
            pltpu.SemaphoreType.DMA,
            pltpu.SemaphoreType.DMA,
        ],
        compiler_params=pltpu.CompilerParams(use_tc_tiling_on_sc=False),
    )
    def k(xs_hbm, e_hbm, out_hbm, deg_hbm, src0_v, dst0_v, src1_v, dst1_v,
          rows0, rows1, rows2, ones_v, zd_v, acc_sh, deg_sh,
          gsem0, gsem1, gsem2, ssem0, ssem1, ssem2, dsem, isem0, isem1):
        cid = lax.axis_index("c")
        sid = lax.axis_index("s")

        zero16 = jnp.zeros((16,), jnp.float32)
        one16 = jnp.ones((16,), jnp.float32)

        @pl.loop(0, CHUNK)
        def _(r):
            ones_v[r, :] = one16
            zd_v[r, :] = zero16

            @pl.loop(0, HALF, step=16)
            def _(cc):
                rows0[r, pl.ds(cc, 16)] = zero16

        @pl.loop(0, ROWS_PER_SUB // ZR)
        def _(kk):
            base = sid * ROWS_PER_SUB + kk * ZR
            pltpu.sync_copy(rows0.at[pl.ds(0, ZR)], acc_sh.at[pl.ds(base, ZR)])
            pltpu.sync_copy(zd_v.at[pl.ds(0, ZR)], deg_sh.at[pl.ds(base, ZR)])

        plsc.subcore_barrier()

        xh = xs_hbm.at[cid]
        rows = (rows0, rows1, rows2)
        gsems = (gsem0, gsem1, gsem2)
        ssems = (ssem0, ssem1, ssem2)
        srcs = (src0_v, src1_v)
        dsts = (dst0_v, dst1_v)
        isems = (isem0, isem1)
        cbase = sid * (NBATCH * IB)
        last_cb = cbase + (NBATCH - 1) * IB

        pltpu.async_copy(e_hbm.at[0, pl.ds(cbase, IB)], src0_v, isem0)
        pltpu.async_copy(e_hbm.at[1, pl.ds(cbase, IB)], dst0_v, isem0)

        @pl.loop(0, NBATCH // 2)
        def _(go):
            for gg in range(2):
                sbuf, dbuf, isem = srcs[gg], dsts[gg], isems[gg]
                pltpu.make_async_copy(
                    e_hbm.at[0, pl.ds(cbase, IB)], sbuf, isem).wait()
                pltpu.make_async_copy(
                    e_hbm.at[1, pl.ds(cbase, IB)], dbuf, isem).wait()
                nxt = jnp.minimum(cbase + (2 * go + gg + 1) * IB, last_cb)
                ngg = (gg + 1) % 2
                pltpu.async_copy(e_hbm.at[0, pl.ds(nxt, IB)],
                                 srcs[ngg], isems[ngg])
                pltpu.async_copy(e_hbm.at[1, pl.ds(nxt, IB)],
                                 dsts[ngg], isems[ngg])

                gat = [None, None, None]
                scats = [None, None, None]
                deg_descs = [[], []]
                gat[0] = pltpu.async_copy(xh.at[sbuf.at[0]], rows0, gsem0)
                gat[1] = pltpu.async_copy(xh.at[sbuf.at[1]], rows1, gsem1)
                for c in range(IB):
                    b = c % 3
                    gat[b].wait()
                    if c + 2 < IB:
                        nb = (c + 2) % 3
                        if scats[nb] is not None:
                            scats[nb].wait()
                        gat[nb] = pltpu.async_copy(
                            xh.at[sbuf.at[c + 2]], rows[nb], gsems[nb])
                    scats[b] = pltpu.async_copy(
                        rows[b], acc_sh.at[dbuf.at[c]], ssems[b], add=True)
                    par = c % 2

                    @pl.when(cid == par)
                    def _():
                        deg_descs[par].append(pltpu.async_copy(
                            ones_v, deg_sh.at[dbuf.at[c]], dsem, add=True))
                for sc in scats:
                    if sc is not None:
                        sc.wait()
                for par in range(2):
                    @pl.when(cid == par)
                    def _():
                        for dd in deg_descs[par]:
                            dd.wait()

        pltpu.make_async_copy(e_hbm.at[0, pl.ds(cbase, IB)], src0_v, isem0).wait()
        pltpu.make_async_copy(e_hbm.at[1, pl.ds(cbase, IB)], dst0_v, isem0).wait()

        plsc.subcore_barrier()

        base = sid * ROWS_PER_SUB
        pltpu.sync_copy(acc_sh.at[pl.ds(base, ROWS_PER_SUB)],
                        out_hbm.at[cid, pl.ds(base, ROWS_PER_SUB)])
        pltpu.sync_copy(deg_sh.at[pl.ds(base, ROWS_PER_SUB)],
                        deg_hbm.at[cid, pl.ds(base, ROWS_PER_SUB)])

    return k(xs, edges)


BLK = 2000


def _tc_combine(x, part, degp, W_self, W_neigh, b, gamma, beta):
    def body(x_ref, p_ref, d_ref, ws_ref, wn_ref, b_ref, g_ref, be_ref, o_ref):
        xb = x_ref[...]
        psum = jnp.concatenate([p_ref[0], p_ref[1]], axis=1)
        deg = d_ref[0, :, 0:1] + d_ref[1, :, 0:1]
        agg = psum / jnp.maximum(deg, 1.0)
        h = jnp.dot(xb, ws_ref[...], preferred_element_type=jnp.float32)
        h = h + jnp.dot(agg, wn_ref[...], preferred_element_type=jnp.float32)
        h = h + b_ref[...]
        mu = jnp.mean(h, axis=1, keepdims=True)
        var = jnp.mean((h - mu) * (h - mu), axis=1, keepdims=True)
        h = (h - mu) * lax.rsqrt(var + 1e-5) * g_ref[...] + be_ref[...]
        o_ref[...] = jnp.maximum(h, 0.0) + xb

    return pl.pallas_call(
        body,
        grid=(N // BLK,),
        in_specs=[
            pl.BlockSpec((BLK, D), lambda i: (i, 0)),
            pl.BlockSpec((NC, BLK, HALF), lambda i: (0, i, 0)),
            pl.BlockSpec((NC, BLK, DEGW), lambda i: (0, i, 0)),
            pl.BlockSpec((D, D), lambda i: (0, 0)),
            pl.BlockSpec((D, D), lambda i: (0, 0)),
            pl.BlockSpec((1, D), lambda i: (0, 0)),
            pl.BlockSpec((1, D), lambda i: (0, 0)),
            pl.BlockSpec((1, D), lambda i: (0, 0)),
        ],
        out_specs=pl.BlockSpec((BLK, D), lambda i: (i, 0)),
        out_shape=jax.ShapeDtypeStruct((N, D), jnp.float32),
    )(x, part, degp, W_self, W_neigh, b, gamma, beta)


@jax.jit
def kernel(x, edge_index, W_self, W_neigh, b, gamma, beta):
    xs = jnp.stack([x[:, :HALF], x[:, HALF:]])
    edges = edge_index.reshape(2, NCHTOT, CHUNK)
    part, degp = _sc_aggregate(xs, edges)
    return _tc_combine(
        x, part, degp, W_self, W_neigh,
        b.reshape(1, D), gamma.reshape(1, D), beta.reshape(1, D),
    )


# speedup vs baseline: 12.4430x; 1.0075x over previous
import functools

import jax
import jax.numpy as jnp
from jax import lax
from jax.experimental import pallas as pl
from jax.experimental.pallas import tpu as pltpu
from jax.experimental.pallas import tpu_sc as plsc

N, E, D = 10000, 320000, 128
HALF = D // 2
NC, NS = 2, 16
EPT = E // NS
CHUNK = 125
IB = 20
NBATCH = EPT // (CHUNK * IB)
NCHTOT = E // CHUNK
ZR = 80
NPAD = 10240
ROWS_PER_SUB = NPAD // NS
DEGW = 16


def _sc_aggregate(xs, edges):
    mesh = plsc.VectorSubcoreMesh(
        core_axis_name="c", subcore_axis_name="s", num_cores=NC, num_subcores=NS
    )

    @functools.partial(
        pl.kernel,
        out_type=[
            jax.ShapeDtypeStruct((NC, NPAD, HALF), jnp.float32),
            jax.ShapeDtypeStruct((NC, NPAD, DEGW), jnp.float32),
        ],
        mesh=mesh,
        scratch_types=[
            pltpu.VMEM((IB, CHUNK), jnp.int32),
            pltpu.VMEM((IB, CHUNK), jnp.int32),
            pltpu.VMEM((IB, CHUNK), jnp.int32),
            pltpu.VMEM((IB, CHUNK), jnp.int32),
            pltpu.VMEM((CHUNK, HALF), jnp.float32),
            pltpu.VMEM((CHUNK, HALF), jnp.float32),
            pltpu.VMEM((CHUNK, HALF), jnp.float32),
            pltpu.VMEM((CHUNK, DEGW), jnp.float32),
            pltpu.VMEM((CHUNK, DEGW), jnp.float32),
            pltpu.VMEM_SHARED((NPAD, HALF), jnp.float32),
            pltpu.VMEM_SHARED((NPAD, DEGW), jnp.float32),
            pltpu.SemaphoreType.DMA,
            pltpu.SemaphoreType.DMA,
            pltpu.SemaphoreType.DMA,
            pltpu.SemaphoreType.DMA,
            pltpu.SemaphoreType.DMA,
            pltpu.SemaphoreType.DMA,
            pltpu.SemaphoreType.DMA,
            pltpu.SemaphoreType.DMA,
            pltpu.SemaphoreType.DMA,
        ],
        compiler_params=pltpu.CompilerParams(use_tc_tiling_on_sc=False),
    )
    def k(xs_hbm, e_hbm, out_hbm, deg_hbm, src0_v, dst0_v, src1_v, dst1_v,
          rows0, rows1, rows2, ones_v, zd_v, acc_sh, deg_sh,
          gsem0, gsem1, gsem2, ssem0, ssem1, ssem2, dsem, isem0, isem1):
        cid = lax.axis_index("c")
        sid = lax.axis_index("s")

        zero16 = jnp.zeros((16,), jnp.float32)
        one16 = jnp.ones((16,), jnp.float32)

        @pl.loop(0, CHUNK)
        def _(r):
            ones_v[r, :] = one16
            zd_v[r, :] = zero16

            @pl.loop(0, HALF, step=16)
            def _(cc):
                rows0[r, pl.ds(cc, 16)] = zero16

        @pl.loop(0, ROWS_PER_SUB // ZR)
        def _(kk):
            base = sid * ROWS_PER_SUB + kk * ZR
            pltpu.sync_copy(rows0.at[pl.ds(0, ZR)], acc_sh.at[pl.ds(base, ZR)])
            pltpu.sync_copy(zd_v.at[pl.ds(0, ZR)], deg_sh.at[pl.ds(base, ZR)])

        plsc.subcore_barrier()

        xh = xs_hbm.at[cid]
        rows = (rows0, rows1, rows2)
        gsems = (gsem0, gsem1, gsem2)
        ssems = (ssem0, ssem1, ssem2)
        srcs = (src0_v, src1_v)
        dsts = (dst0_v, dst1_v)
        isems = (isem0, isem1)
        cbase = sid * (NBATCH * IB)
        last_cb = cbase + (NBATCH - 1) * IB

        pltpu.async_copy(e_hbm.at[0, pl.ds(cbase, IB)], src0_v, isem0)
        pltpu.async_copy(e_hbm.at[1, pl.ds(cbase, IB)], dst0_v, isem0)

        @pl.loop(0, NBATCH // 2)
        def _(go):
            for gg in range(2):
                sbuf, dbuf, isem = srcs[gg], dsts[gg], isems[gg]
                pltpu.make_async_copy(
                    e_hbm.at[0, pl.ds(cbase, IB)], sbuf, isem).wait()
                pltpu.make_async_copy(
                    e_hbm.at[1, pl.ds(cbase, IB)], dbuf, isem).wait()
                nxt = jnp.minimum(cbase + (2 * go + gg + 1) * IB, last_cb)
                ngg = (gg + 1) % 2
                pltpu.async_copy(e_hbm.at[0, pl.ds(nxt, IB)],
                                 srcs[ngg], isems[ngg])
                pltpu.async_copy(e_hbm.at[1, pl.ds(nxt, IB)],
                                 dsts[ngg], isems[ngg])

                gat = [None, None, None]
                scats = [None, None, None]
                deg_descs = [[], []]
                gat[0] = pltpu.async_copy(xh.at[sbuf.at[0]], rows0, gsem0)
                gat[1] = pltpu.async_copy(xh.at[sbuf.at[1]], rows1, gsem1)
                for c in range(IB):
                    b = c % 3
                    gat[b].wait()
                    if c + 2 < IB:
                        nb = (c + 2) % 3
                        if scats[nb] is not None:
                            scats[nb].wait()
                        gat[nb] = pltpu.async_copy(
                            xh.at[sbuf.at[c + 2]], rows[nb], gsems[nb])
                    scats[b] = pltpu.async_copy(
                        rows[b], acc_sh.at[dbuf.at[c]], ssems[b], add=True)
                    par = c % 2

                    @pl.when(cid == par)
                    def _():
                        deg_descs[par].append(pltpu.async_copy(
                            ones_v, deg_sh.at[dbuf.at[c]], dsem, add=True))
                for sc in scats:
                    if sc is not None:
                        sc.wait()
                for par in range(2):
                    @pl.when(cid == par)
                    def _():
                        for dd in deg_descs[par]:
                            dd.wait()

        pltpu.make_async_copy(e_hbm.at[0, pl.ds(cbase, IB)], src0_v, isem0).wait()
        pltpu.make_async_copy(e_hbm.at[1, pl.ds(cbase, IB)], dst0_v, isem0).wait()

        plsc.subcore_barrier()

        base = sid * ROWS_PER_SUB
        pltpu.sync_copy(acc_sh.at[pl.ds(base, ROWS_PER_SUB)],
                        out_hbm.at[cid, pl.ds(base, ROWS_PER_SUB)])
        pltpu.sync_copy(deg_sh.at[pl.ds(base, ROWS_PER_SUB)],
                        deg_hbm.at[cid, pl.ds(base, ROWS_PER_SUB)])

    return k(xs, edges)


BLK = 2000


def _tc_combine(x, part, degp, W_self, W_neigh, b, gamma, beta):
    def body(x_ref, p_ref, d_ref, ws_ref, wn_ref, b_ref, g_ref, be_ref, o_ref):
        xb = x_ref[...]
        psum = jnp.concatenate([p_ref[0], p_ref[1]], axis=1)
        deg = d_ref[0, :, 0:1] + d_ref[1, :, 0:1]
        agg = psum / jnp.maximum(deg, 1.0)
        h = jnp.dot(xb, ws_ref[...], preferred_element_type=jnp.float32)
        h = h + jnp.dot(agg, wn_ref[...], preferred_element_type=jnp.float32)
        h = h + b_ref[...]
        mu = jnp.mean(h, axis=1, keepdims=True)
        var = jnp.mean((h - mu) * (h - mu), axis=1, keepdims=True)
        h = (h - mu) * lax.rsqrt(var + 1e-5) * g_ref[...] + be_ref[...]
        o_ref[...] = jnp.maximum(h, 0.0) + xb

    return pl.pallas_call(
        body,
        grid=(N // BLK,),
        in_specs=[
            pl.BlockSpec((BLK, D), lambda i: (i, 0)),
            pl.BlockSpec((NC, BLK, HALF), lambda i: (0, i, 0)),
            pl.BlockSpec((NC, BLK, DEGW), lambda i: (0, i, 0)),
            pl.BlockSpec((D, D), lambda i: (0, 0)),
            pl.BlockSpec((D, D), lambda i: (0, 0)),
            pl.BlockSpec((1, D), lambda i: (0, 0)),
            pl.BlockSpec((1, D), lambda i: (0, 0)),
            pl.BlockSpec((1, D), lambda i: (0, 0)),
        ],
        out_specs=pl.BlockSpec((BLK, D), lambda i: (i, 0)),
        out_shape=jax.ShapeDtypeStruct((N, D), jnp.float32),
    )(x, part, degp, W_self, W_neigh, b, gamma, beta)


@jax.jit
def kernel(x, edge_index, W_self, W_neigh, b, gamma, beta):
    xs = jnp.stack([x[:, :HALF], x[:, HALF:]])
    edges = edge_index.reshape(2, NCHTOT, CHUNK)
    part, degp = _sc_aggregate(xs, edges)
    return _tc_combine(
        x, part, degp, W_self, W_neigh,
        b.reshape(1, D), gamma.reshape(1, D), beta.reshape(1, D),
    )


# speedup vs baseline: 12.4591x; 1.0013x over previous
import functools

import jax
import jax.numpy as jnp
from jax import lax
from jax.experimental import pallas as pl
from jax.experimental.pallas import tpu as pltpu
from jax.experimental.pallas import tpu_sc as plsc

N, E, D = 10000, 320000, 128
HALF = D // 2
NC, NS = 2, 16
EPT = E // NS
CHUNK = 125
IB = 10
NBATCH = EPT // (CHUNK * IB)
NCHTOT = E // CHUNK
ZR = 125
NPAD = 10000
ROWS_PER_SUB = NPAD // NS
DEGW = 16


def _sc_aggregate(xs, edges):
    mesh = plsc.VectorSubcoreMesh(
        core_axis_name="c", subcore_axis_name="s", num_cores=NC, num_subcores=NS
    )

    @functools.partial(
        pl.kernel,
        out_type=[
            jax.ShapeDtypeStruct((NC, NPAD, HALF), jnp.float32),
            jax.ShapeDtypeStruct((NC, NPAD, DEGW), jnp.float32),
        ],
        mesh=mesh,
        scratch_types=[
            pltpu.VMEM((IB, CHUNK), jnp.int32),
            pltpu.VMEM((IB, CHUNK), jnp.int32),
            pltpu.VMEM((IB, CHUNK), jnp.int32),
            pltpu.VMEM((IB, CHUNK), jnp.int32),
            pltpu.VMEM((CHUNK, HALF), jnp.float32),
            pltpu.VMEM((CHUNK, HALF), jnp.float32),
            pltpu.VMEM((CHUNK, HALF), jnp.float32),
            pltpu.VMEM((CHUNK, HALF), jnp.float32),
            pltpu.VMEM((CHUNK, DEGW), jnp.float32),
            pltpu.VMEM((CHUNK, DEGW), jnp.float32),
            pltpu.VMEM_SHARED((NPAD, HALF), jnp.float32),
            pltpu.VMEM_SHARED((NPAD, DEGW), jnp.float32),
            pltpu.SemaphoreType.DMA,
            pltpu.SemaphoreType.DMA,
            pltpu.SemaphoreType.DMA,
            pltpu.SemaphoreType.DMA,
            pltpu.SemaphoreType.DMA,
            pltpu.SemaphoreType.DMA,
            pltpu.SemaphoreType.DMA,
            pltpu.SemaphoreType.DMA,
            pltpu.SemaphoreType.DMA,
            pltpu.SemaphoreType.DMA,
            pltpu.SemaphoreType.DMA,
        ],
        compiler_params=pltpu.CompilerParams(use_tc_tiling_on_sc=False),
    )
    def k(xs_hbm, e_hbm, out_hbm, deg_hbm, src0_v, dst0_v, src1_v, dst1_v,
          rows0, rows1, rows2, rows3, ones_v, zd_v, acc_sh, deg_sh,
          gsem0, gsem1, gsem2, gsem3, ssem0, ssem1, ssem2, ssem3,
          dsem, isem0, isem1):
        cid = lax.axis_index("c")
        sid = lax.axis_index("s")

        zero16 = jnp.zeros((16,), jnp.float32)
        one16 = jnp.ones((16,), jnp.float32)

        @pl.loop(0, CHUNK)
        def _(r):
            ones_v[r, :] = one16
            zd_v[r, :] = zero16

            @pl.loop(0, HALF, step=16)
            def _(cc):
                rows0[r, pl.ds(cc, 16)] = zero16

        @pl.loop(0, ROWS_PER_SUB // ZR)
        def _(kk):
            base = sid * ROWS_PER_SUB + kk * ZR
            pltpu.sync_copy(rows0.at[pl.ds(0, ZR)], acc_sh.at[pl.ds(base, ZR)])
            pltpu.sync_copy(zd_v.at[pl.ds(0, ZR)], deg_sh.at[pl.ds(base, ZR)])

        plsc.subcore_barrier()

        xh = xs_hbm.at[cid]
        rows = (rows0, rows1, rows2, rows3)
        gsems = (gsem0, gsem1, gsem2, gsem3)
        ssems = (ssem0, ssem1, ssem2, ssem3)
        NB = 4
        srcs = (src0_v, src1_v)
        dsts = (dst0_v, dst1_v)
        isems = (isem0, isem1)
        cbase = sid * (NBATCH * IB)
        last_cb = cbase + (NBATCH - 1) * IB

        pltpu.async_copy(e_hbm.at[0, pl.ds(cbase, IB)], src0_v, isem0)
        pltpu.async_copy(e_hbm.at[1, pl.ds(cbase, IB)], dst0_v, isem0)

        @pl.loop(0, NBATCH // 2)
        def _(go):
            for gg in range(2):
                sbuf, dbuf, isem = srcs[gg], dsts[gg], isems[gg]
                pltpu.make_async_copy(
                    e_hbm.at[0, pl.ds(cbase, IB)], sbuf, isem).wait()
                pltpu.make_async_copy(
                    e_hbm.at[1, pl.ds(cbase, IB)], dbuf, isem).wait()
                nxt = jnp.minimum(cbase + (2 * go + gg + 1) * IB, last_cb)
                ngg = (gg + 1) % 2
                pltpu.async_copy(e_hbm.at[0, pl.ds(nxt, IB)],
                                 srcs[ngg], isems[ngg])
                pltpu.async_copy(e_hbm.at[1, pl.ds(nxt, IB)],
                                 dsts[ngg], isems[ngg])

                gat = [None] * NB
                scats = [None] * NB
                deg_descs = [[], []]
                for p in range(NB - 1):
                    gat[p] = pltpu.async_copy(
                        xh.at[sbuf.at[p]], rows[p], gsems[p])
                for c in range(IB):
                    b = c % NB
                    gat[b].wait()
                    if c + NB - 1 < IB:
                        nb = (c + NB - 1) % NB
                        if scats[nb] is not None:
                            scats[nb].wait()
                        gat[nb] = pltpu.async_copy(
                            xh.at[sbuf.at[c + NB - 1]], rows[nb], gsems[nb])
                    scats[b] = pltpu.async_copy(
                        rows[b], acc_sh.at[dbuf.at[c]], ssems[b], add=True)
                    par = c % 2

                    @pl.when(cid == par)
                    def _():
                        deg_descs[par].append(pltpu.async_copy(
                            ones_v, deg_sh.at[dbuf.at[c]], dsem, add=True))
                for sc in scats:
                    if sc is not None:
                        sc.wait()
                for par in range(2):
                    @pl.when(cid == par)
                    def _():
                        for dd in deg_descs[par]:
                            dd.wait()

        pltpu.make_async_copy(e_hbm.at[0, pl.ds(cbase, IB)], src0_v, isem0).wait()
        pltpu.make_async_copy(e_hbm.at[1, pl.ds(cbase, IB)], dst0_v, isem0).wait()

        plsc.subcore_barrier()

        base = sid * ROWS_PER_SUB
        pltpu.sync_copy(acc_sh.at[pl.ds(base, ROWS_PER_SUB)],
                        out_hbm.at[cid, pl.ds(base, ROWS_PER_SUB)])
        pltpu.sync_copy(deg_sh.at[pl.ds(base, ROWS_PER_SUB)],
                        deg_hbm.at[cid, pl.ds(base, ROWS_PER_SUB)])

    return k(xs, edges)


BLK = 2000


def _tc_combine(x, part, degp, W_self, W_neigh, b, gamma, beta):
    def body(x_ref, p_ref, d_ref, ws_ref, wn_ref, b_ref, g_ref, be_ref, o_ref):
        xb = x_ref[...]
        psum = jnp.concatenate([p_ref[0], p_ref[1]], axis=1)
        deg = d_ref[0, :, 0:1] + d_ref[1, :, 0:1]
        agg = psum / jnp.maximum(deg, 1.0)
        h = jnp.dot(xb, ws_ref[...], preferred_element_type=jnp.float32)
        h = h + jnp.dot(agg, wn_ref[...], preferred_element_type=jnp.float32)
        h = h + b_ref[...]
        mu = jnp.mean(h, axis=1, keepdims=True)
        var = jnp.mean((h - mu) * (h - mu), axis=1, keepdims=True)
        h = (h - mu) * lax.rsqrt(var + 1e-5) * g_ref[...] + be_ref[...]
        o_ref[...] = jnp.maximum(h, 0.0) + xb

    return pl.pallas_call(
        body,
        grid=(N // BLK,),
        in_specs=[
            pl.BlockSpec((BLK, D), lambda i: (i, 0)),
            pl.BlockSpec((NC, BLK, HALF), lambda i: (0, i, 0)),
            pl.BlockSpec((NC, BLK, DEGW), lambda i: (0, i, 0)),
            pl.BlockSpec((D, D), lambda i: (0, 0)),
            pl.BlockSpec((D, D), lambda i: (0, 0)),
            pl.BlockSpec((1, D), lambda i: (0, 0)),
            pl.BlockSpec((1, D), lambda i: (0, 0)),
            pl.BlockSpec((1, D), lambda i: (0, 0)),
        ],
        out_specs=pl.BlockSpec((BLK, D), lambda i: (i, 0)),
        out_shape=jax.ShapeDtypeStruct((N, D), jnp.float32),
    )(x, part, degp, W_self, W_neigh, b, gamma, beta)


@jax.jit
def kernel(x, edge_index, W_self, W_neigh, b, gamma, beta):
    xs = jnp.stack([x[:, :HALF], x[:, HALF:]])
    edges = edge_index.reshape(2, NCHTOT, CHUNK)
    part, degp = _sc_aggregate(xs, edges)
    return _tc_combine(
        x, part, degp, W_self, W_neigh,
        b.reshape(1, D), gamma.reshape(1, D), beta.reshape(1, D),
    )


# speedup vs baseline: 15.9986x; 1.2841x over previous
import functools

import jax
import jax.numpy as jnp
from jax import lax
from jax.experimental import pallas as pl
from jax.experimental.pallas import tpu as pltpu
from jax.experimental.pallas import tpu_sc as plsc

N, E, D = 10000, 320000, 128
HALF = D // 2
NC, NS = 2, 16
EPT = E // NS
CHUNK = 125
IB = 10
NBATCH = EPT // (CHUNK * IB)
NCHTOT = E // CHUNK
ZR = 125
NPAD = 10000
ROWS_PER_SUB = NPAD // NS
DEGW = 16


def _sc_aggregate(xs, edges):
    mesh = plsc.VectorSubcoreMesh(
        core_axis_name="c", subcore_axis_name="s", num_cores=NC, num_subcores=NS
    )

    @functools.partial(
        pl.kernel,
        out_type=[
            jax.ShapeDtypeStruct((NC, NPAD, HALF), jnp.bfloat16),
            jax.ShapeDtypeStruct((NC, NPAD, DEGW), jnp.float32),
        ],
        mesh=mesh,
        scratch_types=[
            pltpu.VMEM((IB, CHUNK), jnp.int32),
            pltpu.VMEM((IB, CHUNK), jnp.int32),
            pltpu.VMEM((IB, CHUNK), jnp.int32),
            pltpu.VMEM((IB, CHUNK), jnp.int32),
            pltpu.VMEM((CHUNK, HALF), jnp.bfloat16),
            pltpu.VMEM((CHUNK, HALF), jnp.bfloat16),
            pltpu.VMEM((CHUNK, HALF), jnp.bfloat16),
            pltpu.VMEM((CHUNK, HALF), jnp.bfloat16),
            pltpu.VMEM((CHUNK, DEGW), jnp.float32),
            pltpu.VMEM((CHUNK, DEGW), jnp.float32),
            pltpu.VMEM_SHARED((NPAD, HALF), jnp.bfloat16),
            pltpu.VMEM_SHARED((NPAD, DEGW), jnp.float32),
            pltpu.SemaphoreType.DMA,
            pltpu.SemaphoreType.DMA,
            pltpu.SemaphoreType.DMA,
            pltpu.SemaphoreType.DMA,
            pltpu.SemaphoreType.DMA,
            pltpu.SemaphoreType.DMA,
            pltpu.SemaphoreType.DMA,
            pltpu.SemaphoreType.DMA,
            pltpu.SemaphoreType.DMA,
            pltpu.SemaphoreType.DMA,
            pltpu.SemaphoreType.DMA,
        ],
        compiler_params=pltpu.CompilerParams(use_tc_tiling_on_sc=False),
    )
    def k(xs_hbm, e_hbm, out_hbm, deg_hbm, src0_v, dst0_v, src1_v, dst1_v,
          rows0, rows1, rows2, rows3, ones_v, zd_v, acc_sh, deg_sh,
          gsem0, gsem1, gsem2, gsem3, ssem0, ssem1, ssem2, ssem3,
          dsem, isem0, isem1):
        cid = lax.axis_index("c")
        sid = lax.axis_index("s")

        zero16 = jnp.zeros((16,), jnp.float32)
        one16 = jnp.ones((16,), jnp.float32)
        zero32b = jnp.zeros((32,), jnp.bfloat16)

        @pl.loop(0, CHUNK)
        def _(r):
            ones_v[r, :] = one16
            zd_v[r, :] = zero16

            @pl.loop(0, HALF, step=32)
            def _(cc):
                rows0[r, pl.ds(cc, 32)] = zero32b

        @pl.loop(0, ROWS_PER_SUB // ZR)
        def _(kk):
            base = sid * ROWS_PER_SUB + kk * ZR
            pltpu.sync_copy(rows0.at[pl.ds(0, ZR)], acc_sh.at[pl.ds(base, ZR)])
            pltpu.sync_copy(zd_v.at[pl.ds(0, ZR)], deg_sh.at[pl.ds(base, ZR)])

        plsc.subcore_barrier()

        xh = xs_hbm.at[cid]
        rows = (rows0, rows1, rows2, rows3)
        gsems = (gsem0, gsem1, gsem2, gsem3)
        ssems = (ssem0, ssem1, ssem2, ssem3)
        NB = 4
        srcs = (src0_v, src1_v)
        dsts = (dst0_v, dst1_v)
        isems = (isem0, isem1)
        cbase = sid * (NBATCH * IB)
        last_cb = cbase + (NBATCH - 1) * IB

        pltpu.async_copy(e_hbm.at[0, pl.ds(cbase, IB)], src0_v, isem0)
        pltpu.async_copy(e_hbm.at[1, pl.ds(cbase, IB)], dst0_v, isem0)

        @pl.loop(0, NBATCH // 2)
        def _(go):
            for gg in range(2):
                sbuf, dbuf, isem = srcs[gg], dsts[gg], isems[gg]
                pltpu.make_async_copy(
                    e_hbm.at[0, pl.ds(cbase, IB)], sbuf, isem).wait()
                pltpu.make_async_copy(
                    e_hbm.at[1, pl.ds(cbase, IB)], dbuf, isem).wait()
                nxt = jnp.minimum(cbase + (2 * go + gg + 1) * IB, last_cb)
                ngg = (gg + 1) % 2
                pltpu.async_copy(e_hbm.at[0, pl.ds(nxt, IB)],
                                 srcs[ngg], isems[ngg])
                pltpu.async_copy(e_hbm.at[1, pl.ds(nxt, IB)],
                                 dsts[ngg], isems[ngg])

                gat = [None] * NB
                scats = [None] * NB
                deg_descs = [[], []]
                for p in range(NB - 1):
                    gat[p] = pltpu.async_copy(
                        xh.at[sbuf.at[p]], rows[p], gsems[p])
                for c in range(IB):
                    b = c % NB
                    gat[b].wait()
                    if c + NB - 1 < IB:
                        nb = (c + NB - 1) % NB
                        if scats[nb] is not None:
                            scats[nb].wait()
                        gat[nb] = pltpu.async_copy(
                            xh.at[sbuf.at[c + NB - 1]], rows[nb], gsems[nb])
                    scats[b] = pltpu.async_copy(
                        rows[b], acc_sh.at[dbuf.at[c]], ssems[b], add=True)
                    par = c % 2

                    @pl.when(cid == par)
                    def _():
                        deg_descs[par].append(pltpu.async_copy(
                            ones_v, deg_sh.at[dbuf.at[c]], dsem, add=True))
                for sc in scats:
                    if sc is not None:
                        sc.wait()
                for par in range(2):
                    @pl.when(cid == par)
                    def _():
                        for dd in deg_descs[par]:
                            dd.wait()

        pltpu.make_async_copy(e_hbm.at[0, pl.ds(cbase, IB)], src0_v, isem0).wait()
        pltpu.make_async_copy(e_hbm.at[1, pl.ds(cbase, IB)], dst0_v, isem0).wait()

        plsc.subcore_barrier()

        base = sid * ROWS_PER_SUB
        pltpu.sync_copy(acc_sh.at[pl.ds(base, ROWS_PER_SUB)],
                        out_hbm.at[cid, pl.ds(base, ROWS_PER_SUB)])
        pltpu.sync_copy(deg_sh.at[pl.ds(base, ROWS_PER_SUB)],
                        deg_hbm.at[cid, pl.ds(base, ROWS_PER_SUB)])

    return k(xs, edges)


BLK = 2000


def _tc_combine(x, part, degp, W_self, W_neigh, b, gamma, beta):
    def body(x_ref, p_ref, d_ref, ws_ref, wn_ref, b_ref, g_ref, be_ref, o_ref):
        xb = x_ref[...]
        psum = jnp.concatenate([p_ref[0], p_ref[1]], axis=1).astype(jnp.float32)
        deg = d_ref[0, :, 0:1] + d_ref[1, :, 0:1]
        agg = psum / jnp.maximum(deg, 1.0)
        h = jnp.dot(xb, ws_ref[...], preferred_element_type=jnp.float32)
        h = h + jnp.dot(agg, wn_ref[...], preferred_element_type=jnp.float32)
        h = h + b_ref[...]
        mu = jnp.mean(h, axis=1, keepdims=True)
        var = jnp.mean((h - mu) * (h - mu), axis=1, keepdims=True)
        h = (h - mu) * lax.rsqrt(var + 1e-5) * g_ref[...] + be_ref[...]
        o_ref[...] = jnp.maximum(h, 0.0) + xb

    return pl.pallas_call(
        body,
        grid=(N // BLK,),
        in_specs=[
            pl.BlockSpec((BLK, D), lambda i: (i, 0)),
            pl.BlockSpec((NC, BLK, HALF), lambda i: (0, i, 0)),
            pl.BlockSpec((NC, BLK, DEGW), lambda i: (0, i, 0)),
            pl.BlockSpec((D, D), lambda i: (0, 0)),
            pl.BlockSpec((D, D), lambda i: (0, 0)),
            pl.BlockSpec((1, D), lambda i: (0, 0)),
            pl.BlockSpec((1, D), lambda i: (0, 0)),
            pl.BlockSpec((1, D), lambda i: (0, 0)),
        ],
        out_specs=pl.BlockSpec((BLK, D), lambda i: (i, 0)),
        out_shape=jax.ShapeDtypeStruct((N, D), jnp.float32),
    )(x, part, degp, W_self, W_neigh, b, gamma, beta)


@jax.jit
def kernel(x, edge_index, W_self, W_neigh, b, gamma, beta):
    xs = jnp.stack([x[:, :HALF], x[:, HALF:]]).astype(jnp.bfloat16)
    edges = edge_index.reshape(2, NCHTOT, CHUNK)
    part, degp = _sc_aggregate(xs, edges)
    return _tc_combine(
        x, part, degp, W_self, W_neigh,
        b.reshape(1, D), gamma.reshape(1, D), beta.reshape(1, D),
    )


# speedup vs baseline: 16.8079x; 1.0506x over previous
import functools

import jax
import jax.numpy as jnp
from jax import lax
from jax.experimental import pallas as pl
from jax.experimental.pallas import tpu as pltpu
from jax.experimental.pallas import tpu_sc as plsc

N, E, D = 10000, 320000, 128
HALF = D // 2
NC, NS = 2, 16
EPT = E // NS
CHUNK = 250
IB = 10
NBATCH = EPT // (CHUNK * IB)
NCHTOT = E // CHUNK
ZR = 125
NPAD = 10000
ROWS_PER_SUB = NPAD // NS
DEGW = 16


def _sc_aggregate(xs, edges):
    mesh = plsc.VectorSubcoreMesh(
        core_axis_name="c", subcore_axis_name="s", num_cores=NC, num_subcores=NS
    )

    @functools.partial(
        pl.kernel,
        out_type=[
            jax.ShapeDtypeStruct((NC, NPAD, HALF), jnp.bfloat16),
            jax.ShapeDtypeStruct((NC, NPAD, DEGW), jnp.float32),
        ],
        mesh=mesh,
        scratch_types=[
            pltpu.VMEM((IB, CHUNK), jnp.int32),
            pltpu.VMEM((IB, CHUNK), jnp.int32),
            pltpu.VMEM((IB, CHUNK), jnp.int32),
            pltpu.VMEM((IB, CHUNK), jnp.int32),
            pltpu.VMEM((CHUNK, HALF), jnp.bfloat16),
            pltpu.VMEM((CHUNK, HALF), jnp.bfloat16),
            pltpu.VMEM((CHUNK, HALF), jnp.bfloat16),
            pltpu.VMEM((CHUNK, HALF), jnp.bfloat16),
            pltpu.VMEM((CHUNK, DEGW), jnp.float32),
            pltpu.VMEM((CHUNK, DEGW), jnp.float32),
            pltpu.VMEM_SHARED((NPAD, HALF), jnp.bfloat16),
            pltpu.VMEM_SHARED((NPAD, DEGW), jnp.float32),
            pltpu.SemaphoreType.DMA,
            pltpu.SemaphoreType.DMA,
            pltpu.SemaphoreType.DMA,
            pltpu.SemaphoreType.DMA,
            pltpu.SemaphoreType.DMA,
            pltpu.SemaphoreType.DMA,
            pltpu.SemaphoreType.DMA,
            pltpu.SemaphoreType.DMA,
            pltpu.SemaphoreType.DMA,
            pltpu.SemaphoreType.DMA,
            pltpu.SemaphoreType.DMA,
        ],
        compiler_params=pltpu.CompilerParams(use_tc_tiling_on_sc=False),
    )
    def k(xs_hbm, e_hbm, out_hbm, deg_hbm, src0_v, dst0_v, src1_v, dst1_v,
          rows0, rows1, rows2, rows3, ones_v, zd_v, acc_sh, deg_sh,
          gsem0, gsem1, gsem2, gsem3, ssem0, ssem1, ssem2, ssem3,
          dsem, isem0, isem1):
        cid = lax.axis_index("c")
        sid = lax.axis_index("s")

        zero16 = jnp.zeros((16,), jnp.float32)
        one16 = jnp.ones((16,), jnp.float32)
        zero32b = jnp.zeros((32,), jnp.bfloat16)

        @pl.loop(0, CHUNK)
        def _(r):
            ones_v[r, :] = one16
            zd_v[r, :] = zero16

            @pl.loop(0, HALF, step=32)
            def _(cc):
                rows0[r, pl.ds(cc, 32)] = zero32b

        @pl.loop(0, ROWS_PER_SUB // ZR)
        def _(kk):
            base = sid * ROWS_PER_SUB + kk * ZR
            pltpu.sync_copy(rows0.at[pl.ds(0, ZR)], acc_sh.at[pl.ds(base, ZR)])
            pltpu.sync_copy(zd_v.at[pl.ds(0, ZR)], deg_sh.at[pl.ds(base, ZR)])

        plsc.subcore_barrier()

        xh = xs_hbm.at[cid]
        rows = (rows0, rows1, rows2, rows3)
        gsems = (gsem0, gsem1, gsem2, gsem3)
        ssems = (ssem0, ssem1, ssem2, ssem3)
        NB = 4
        srcs = (src0_v, src1_v)
        dsts = (dst0_v, dst1_v)
        isems = (isem0, isem1)
        cbase = sid * (NBATCH * IB)
        last_cb = cbase + (NBATCH - 1) * IB

        pltpu.async_copy(e_hbm.at[0, pl.ds(cbase, IB)], src0_v, isem0)
        pltpu.async_copy(e_hbm.at[1, pl.ds(cbase, IB)], dst0_v, isem0)

        @pl.loop(0, NBATCH // 2)
        def _(go):
            for gg in range(2):
                sbuf, dbuf, isem = srcs[gg], dsts[gg], isems[gg]
                pltpu.make_async_copy(
                    e_hbm.at[0, pl.ds(cbase, IB)], sbuf, isem).wait()
                pltpu.make_async_copy(
                    e_hbm.at[1, pl.ds(cbase, IB)], dbuf, isem).wait()
                nxt = jnp.minimum(cbase + (2 * go + gg + 1) * IB, last_cb)
                ngg = (gg + 1) % 2
                pltpu.async_copy(e_hbm.at[0, pl.ds(nxt, IB)],
                                 srcs[ngg], isems[ngg])
                pltpu.async_copy(e_hbm.at[1, pl.ds(nxt, IB)],
                                 dsts[ngg], isems[ngg])

                gat = [None] * NB
                scats = [None] * NB
                deg_descs = [[], []]
                for p in range(NB - 1):
                    gat[p] = pltpu.async_copy(
                        xh.at[sbuf.at[p]], rows[p], gsems[p])
                for c in range(IB):
                    b = c % NB
                    gat[b].wait()
                    if c + NB - 1 < IB:
                        nb = (c + NB - 1) % NB
                        if scats[nb] is not None:
                            scats[nb].wait()
                        gat[nb] = pltpu.async_copy(
                            xh.at[sbuf.at[c + NB - 1]], rows[nb], gsems[nb])
                    scats[b] = pltpu.async_copy(
                        rows[b], acc_sh.at[dbuf.at[c]], ssems[b], add=True)
                    par = c % 2

                    @pl.when(cid == par)
                    def _():
                        deg_descs[par].append(pltpu.async_copy(
                            ones_v, deg_sh.at[dbuf.at[c]], dsem, add=True))
                for sc in scats:
                    if sc is not None:
                        sc.wait()
                for par in range(2):
                    @pl.when(cid == par)
                    def _():
                        for dd in deg_descs[par]:
                            dd.wait()

        pltpu.make_async_copy(e_hbm.at[0, pl.ds(cbase, IB)], src0_v, isem0).wait()
        pltpu.make_async_copy(e_hbm.at[1, pl.ds(cbase, IB)], dst0_v, isem0).wait()

        plsc.subcore_barrier()

        base = sid * ROWS_PER_SUB
        pltpu.sync_copy(acc_sh.at[pl.ds(base, ROWS_PER_SUB)],
                        out_hbm.at[cid, pl.ds(base, ROWS_PER_SUB)])
        pltpu.sync_copy(deg_sh.at[pl.ds(base, ROWS_PER_SUB)],
                        deg_hbm.at[cid, pl.ds(base, ROWS_PER_SUB)])

    return k(xs, edges)


BLK = 2000


def _tc_combine(x, part, degp, W_self, W_neigh, b, gamma, beta):
    def body(x_ref, p_ref, d_ref, ws_ref, wn_ref, b_ref, g_ref, be_ref, o_ref):
        xb = x_ref[...]
        psum = jnp.concatenate([p_ref[0], p_ref[1]], axis=1).astype(jnp.float32)
        deg = d_ref[0, :, 0:1] + d_ref[1, :, 0:1]
        agg = psum / jnp.maximum(deg, 1.0)
        h = jnp.dot(xb, ws_ref[...], preferred_element_type=jnp.float32)
        h = h + jnp.dot(agg, wn_ref[...], preferred_element_type=jnp.float32)
        h = h + b_ref[...]
        mu = jnp.mean(h, axis=1, keepdims=True)
        var = jnp.mean((h - mu) * (h - mu), axis=1, keepdims=True)
        h = (h - mu) * lax.rsqrt(var + 1e-5) * g_ref[...] + be_ref[...]
        o_ref[...] = jnp.maximum(h, 0.0) + xb

    return pl.pallas_call(
        body,
        grid=(N // BLK,),
        in_specs=[
            pl.BlockSpec((BLK, D), lambda i: (i, 0)),
            pl.BlockSpec((NC, BLK, HALF), lambda i: (0, i, 0)),
            pl.BlockSpec((NC, BLK, DEGW), lambda i: (0, i, 0)),
            pl.BlockSpec((D, D), lambda i: (0, 0)),
            pl.BlockSpec((D, D), lambda i: (0, 0)),
            pl.BlockSpec((1, D), lambda i: (0, 0)),
            pl.BlockSpec((1, D), lambda i: (0, 0)),
            pl.BlockSpec((1, D), lambda i: (0, 0)),
        ],
        out_specs=pl.BlockSpec((BLK, D), lambda i: (i, 0)),
        out_shape=jax.ShapeDtypeStruct((N, D), jnp.float32),
    )(x, part, degp, W_self, W_neigh, b, gamma, beta)


@jax.jit
def kernel(x, edge_index, W_self, W_neigh, b, gamma, beta):
    xs = jnp.stack([x[:, :HALF], x[:, HALF:]]).astype(jnp.bfloat16)
    edges = edge_index.reshape(2, NCHTOT, CHUNK)
    part, degp = _sc_aggregate(xs, edges)
    return _tc_combine(
        x, part, degp, W_self, W_neigh,
        b.reshape(1, D), gamma.reshape(1, D), beta.reshape(1, D),
    )
